# baseline (speedup 1.0000x reference)
import jax
import jax.numpy as jnp
from jax import lax
from jax.experimental import pallas as pl
from jax.experimental.pallas import tpu as pltpu
from jax.experimental.pallas import tpu_sc as plsc

BLK = 40
NW = 32
PB = 1024
FSPLIT = 40


def _eye(n):
    i32 = jnp.int32
    return (lax.broadcasted_iota(i32, (n, n), 0)
            == lax.broadcasted_iota(i32, (n, n), 1)).astype(jnp.bfloat16)


def _split3(x):
    f32 = jnp.float32
    bf16 = jnp.bfloat16
    hi = x.astype(bf16)
    r1 = x - hi.astype(f32)
    mid = r1.astype(bf16)
    lo = (r1 - mid.astype(f32)).astype(bf16)
    return hi, mid, lo


def _p_block(in0t_ref, in1t_ref, p_ref):
    a = in0t_ref[...]
    b = in1t_ref[...]
    plo = a[0:32, :] * (0.5 * b[0:32, :] + 0.25 * b[32:64, :])
    phi = a[32:64, :] * (0.75 * b[0:32, :] - 0.25 * b[32:64, :])
    pt = jnp.concatenate([plo, phi], axis=0)
    p_ref[...] = pt.T


def _p_kernel(in0t, in1t, blk0, nblk):
    n = in0t.shape[1]
    return pl.pallas_call(
        _p_block,
        grid=(nblk,),
        in_specs=[
            pl.BlockSpec((64, PB), lambda i: (0, i + blk0)),
            pl.BlockSpec((64, PB), lambda i: (0, i + blk0)),
        ],
        out_specs=pl.BlockSpec((PB, 64), lambda i: (i + blk0, 0)),
        out_shape=jax.ShapeDtypeStruct((n, 64), jnp.float32),
    )(in0t, in1t)


def _c_block(in0t_ref, in1t_ref, m_ref, o_ref):
    i32 = jnp.int32
    a = in0t_ref[...]
    b = in1t_ref[...]
    plo = a[0:32, :] * (0.5 * b[0:32, :] + 0.25 * b[32:64, :])
    phi = a[32:64, :] * (0.75 * b[0:32, :] - 0.25 * b[32:64, :])
    pt = jnp.concatenate([plo, phi], axis=0)
    esel = (lax.broadcasted_iota(i32, (64, 1024), 1) // 16
            == lax.broadcasted_iota(i32, (64, 1024), 0)).astype(jnp.float32)
    q = lax.dot_general(pt, esel, (((0,), (0,)), ((), ())),
                        preferred_element_type=jnp.float32)
    t = q * m_ref[...]
    halves = []
    for h in range(2):
        x = t[:, 512 * h:512 * (h + 1)]
        w = 256
        while w >= 16:
            x = x[:, :w] + x[:, w:]
            w //= 2
        halves.append(x)
    o = jnp.concatenate(halves, axis=1)
    acc = None
    eye = _eye(32)
    for part in _split3(o):
        y = lax.dot_general(eye, part, (((1,), (1,)), ((), ())),
                            preferred_element_type=jnp.float32)
        acc = y if acc is None else acc + y
    o_ref[...] = acc


def _c_kernel(in0t, in1t, in2, nblk):
    return pl.pallas_call(
        _c_block,
        grid=(nblk,),
        in_specs=[
            pl.BlockSpec((64, PB), lambda i: (0, i)),
            pl.BlockSpec((64, PB), lambda i: (0, i)),
            pl.BlockSpec((PB, 1024), lambda i: (i, 0)),
        ],
        out_specs=pl.BlockSpec((32, PB), lambda i: (0, i)),
        out_shape=jax.ShapeDtypeStruct((32, nblk * PB), jnp.float32),
    )(in0t, in1t, in2)


def _row_compute(p_v, in2_v, out_v, nrows):
    def row(r, carry):
        p00 = p_v[r, pl.ds(0, 16)]
        p01 = p_v[r, pl.ds(16, 16)]
        p10 = p_v[r, pl.ds(32, 16)]
        p11 = p_v[r, pl.ds(48, 16)]

        acc0 = [None] * 4
        acc1 = [None] * 4
        for u in range(32):
            plo = p00[u] if u < 16 else p01[u - 16]
            phi = p10[u] if u < 16 else p11[u - 16]
            v0 = in2_v[r, pl.ds(u * 16, 16)] * plo
            v1 = in2_v[r, pl.ds(512 + u * 16, 16)] * phi
            j = u % 4
            if u < 4:
                acc0[j] = v0
                acc1[j] = v1
            else:
                acc0[j] = acc0[j] + v0
                acc1[j] = acc1[j] + v1
        out_v[r, pl.ds(0, 16)] = (acc0[0] + acc0[1]) + (acc0[2] + acc0[3])
        out_v[r, pl.ds(16, 16)] = (acc1[0] + acc1[1]) + (acc1[2] + acc1[3])
        return carry

    lax.fori_loop(0, nrows, row, 0)


def _make_tec_body(F, T, EXTRA):
    def tec_body(p_h, in2_h, out_h,
                 pA, mA, oA, pB, mB, oB,
                 sia, sib, soa, sob):
        cid = lax.axis_index("c")
        sid = lax.axis_index("s")
        wid = sid * 2 + cid

        def start_in(t, dp, dm, sem):
            r0 = F + (t * NW + wid) * BLK
            pltpu.make_async_copy(p_h.at[pl.ds(r0, BLK), :], dp, sem).start()
            pltpu.make_async_copy(in2_h.at[pl.ds(r0, BLK), :], dm, sem).start()

        def wait_in(dp, dm, sem):
            pltpu.make_async_copy(p_h.at[pl.ds(0, BLK), :], dp, sem).wait()
            pltpu.make_async_copy(in2_h.at[pl.ds(0, BLK), :], dm, sem).wait()

        def start_out(t, src, sem):
            r0 = (t * NW + wid) * BLK
            pltpu.make_async_copy(src, out_h.at[pl.ds(r0, BLK), :], sem).start()

        def wait_out(src, sem):
            pltpu.make_async_copy(src, out_h.at[pl.ds(0, BLK), :], sem).wait()

        start_in(0, pA, mA, sia)

        def pair(i, carry):
            t0 = 2 * i
            wait_in(pA, mA, sia)
            start_in(t0 + 1, pB, mB, sib)

            @pl.when(i > 0)
            def _():
                wait_out(oA, soa)

            _row_compute(pA, mA, oA, BLK)
            start_out(t0, oA, soa)

            wait_in(pB, mB, sib)

            @pl.when(t0 + 2 < T)
            def _():
                start_in(t0 + 2, pA, mA, sia)

            @pl.when(i > 0)
            def _():
                wait_out(oB, sob)

            _row_compute(pB, mB, oB, BLK)
            start_out(t0 + 1, oB, sob)
            return carry

        lax.fori_loop(0, T // 2, pair, 0)
        wait_out(oA, soa)
        wait_out(oB, sob)

        for e in range(EXTRA):
            @pl.when(wid == (e % NW))
            def _():
                t = T * NW + e
                rg = F + t * BLK
                pltpu.sync_copy(p_h.at[pl.ds(rg, BLK), :], pA)
                pltpu.sync_copy(in2_h.at[pl.ds(rg, BLK), :], mA)
                _row_compute(pA, mA, oA, BLK)
                pltpu.sync_copy(oA, out_h.at[pl.ds(t * BLK, BLK), :])

    return tec_body


def _build_sc(F, nblk):
    nsc = nblk * BLK
    assert F % 8 == 0, F
    T = (nblk // NW) & ~1
    EXTRA = nblk - T * NW
    f32 = jnp.float32
    mesh = plsc.VectorSubcoreMesh(
        core_axis_name="c", subcore_axis_name="s", num_cores=2, num_subcores=16
    )
    return pl.kernel(
        _make_tec_body(F, T, EXTRA),
        out_type=jax.ShapeDtypeStruct((nsc, 32), f32),
        mesh=mesh,
        scratch_types=[
            pltpu.VMEM((BLK, 64), f32),
            pltpu.VMEM((BLK, 1024), f32),
            pltpu.VMEM((BLK, 32), f32),
            pltpu.VMEM((BLK, 64), f32),
            pltpu.VMEM((BLK, 1024), f32),
            pltpu.VMEM((BLK, 32), f32),
            pltpu.SemaphoreType.DMA,
            pltpu.SemaphoreType.DMA,
            pltpu.SemaphoreType.DMA,
            pltpu.SemaphoreType.DMA,
        ],
    )


def kernel(in0, in1, in2):
    n = in0.shape[0]
    f = FSPLIT * PB
    in0t = in0.T
    in1t = in1.T
    nblk_sc = (n - f) // BLK
    nblk1 = min(512, (nblk_sc // 2 // 16) * 16)
    nblk2 = nblk_sc - nblk1
    f2 = f + nblk1 * BLK
    pblk1 = (f2 - f + PB - 1) // PB
    pblk2 = (n - f2 + PB - 1) // PB
    p1 = _p_kernel(in0t, in1t, f // PB, pblk1)
    out_sc1 = _build_sc(f, nblk1)(p1, in2)
    p2 = _p_kernel(in0t, in1t, f2 // PB, pblk2)
    out_sc2 = _build_sc(f2, nblk2)(p2, in2)
    out_tc_t = _c_kernel(in0t, in1t, in2, FSPLIT)
    out_t = jnp.concatenate(
        [out_tc_t, out_sc1.T, out_sc2.T], axis=1)
    return out_t.T


# speedup vs baseline: 1.0506x; 1.0506x over previous
import jax
import jax.numpy as jnp
from jax import lax
from jax.experimental import pallas as pl
from jax.experimental.pallas import tpu as pltpu
from jax.experimental.pallas import tpu_sc as plsc

BLK = 40
NW = 32
PB = 1024
FSPLIT = 50


def _eye(n):
    i32 = jnp.int32
    return (lax.broadcasted_iota(i32, (n, n), 0)
            == lax.broadcasted_iota(i32, (n, n), 1)).astype(jnp.bfloat16)


def _split3(x):
    f32 = jnp.float32
    bf16 = jnp.bfloat16
    hi = x.astype(bf16)
    r1 = x - hi.astype(f32)
    mid = r1.astype(bf16)
    lo = (r1 - mid.astype(f32)).astype(bf16)
    return hi, mid, lo


def _p_block(in0t_ref, in1t_ref, p_ref):
    a = in0t_ref[...]
    b = in1t_ref[...]
    plo = a[0:32, :] * (0.5 * b[0:32, :] + 0.25 * b[32:64, :])
    phi = a[32:64, :] * (0.75 * b[0:32, :] - 0.25 * b[32:64, :])
    pt = jnp.concatenate([plo, phi], axis=0)
    p_ref[...] = pt.T


def _p_kernel(in0t, in1t, blk0, nblk):
    n = in0t.shape[1]
    return pl.pallas_call(
        _p_block,
        grid=(nblk,),
        in_specs=[
            pl.BlockSpec((64, PB), lambda i: (0, i + blk0)),
            pl.BlockSpec((64, PB), lambda i: (0, i + blk0)),
        ],
        out_specs=pl.BlockSpec((PB, 64), lambda i: (i + blk0, 0)),
        out_shape=jax.ShapeDtypeStruct((n, 64), jnp.float32),
    )(in0t, in1t)


def _c_block(in0t_ref, in1t_ref, m_ref, o_ref):
    i32 = jnp.int32
    a = in0t_ref[...]
    b = in1t_ref[...]
    plo = a[0:32, :] * (0.5 * b[0:32, :] + 0.25 * b[32:64, :])
    phi = a[32:64, :] * (0.75 * b[0:32, :] - 0.25 * b[32:64, :])
    pt = jnp.concatenate([plo, phi], axis=0)
    esel = (lax.broadcasted_iota(i32, (64, 1024), 1) // 16
            == lax.broadcasted_iota(i32, (64, 1024), 0)).astype(jnp.float32)
    q = lax.dot_general(pt, esel, (((0,), (0,)), ((), ())),
                        preferred_element_type=jnp.float32)
    t = q * m_ref[...]
    halves = []
    for h in range(2):
        x = t[:, 512 * h:512 * (h + 1)]
        w = 256
        while w >= 16:
            x = x[:, :w] + x[:, w:]
            w //= 2
        halves.append(x)
    o = jnp.concatenate(halves, axis=1)
    acc = None
    eye = _eye(32)
    for part in _split3(o):
        y = lax.dot_general(eye, part, (((1,), (1,)), ((), ())),
                            preferred_element_type=jnp.float32)
        acc = y if acc is None else acc + y
    o_ref[...] = acc


def _c_kernel(in0t, in1t, in2, nblk):
    return pl.pallas_call(
        _c_block,
        grid=(nblk,),
        in_specs=[
            pl.BlockSpec((64, PB), lambda i: (0, i)),
            pl.BlockSpec((64, PB), lambda i: (0, i)),
            pl.BlockSpec((PB, 1024), lambda i: (i, 0)),
        ],
        out_specs=pl.BlockSpec((32, PB), lambda i: (0, i)),
        out_shape=jax.ShapeDtypeStruct((32, nblk * PB), jnp.float32),
    )(in0t, in1t, in2)


def _row_compute(p_v, in2_v, out_v, nrows):
    def row(r, carry):
        p00 = p_v[r, pl.ds(0, 16)]
        p01 = p_v[r, pl.ds(16, 16)]
        p10 = p_v[r, pl.ds(32, 16)]
        p11 = p_v[r, pl.ds(48, 16)]

        acc0 = [None] * 4
        acc1 = [None] * 4
        for u in range(32):
            plo = p00[u] if u < 16 else p01[u - 16]
            phi = p10[u] if u < 16 else p11[u - 16]
            v0 = in2_v[r, pl.ds(u * 16, 16)] * plo
            v1 = in2_v[r, pl.ds(512 + u * 16, 16)] * phi
            j = u % 4
            if u < 4:
                acc0[j] = v0
                acc1[j] = v1
            else:
                acc0[j] = acc0[j] + v0
                acc1[j] = acc1[j] + v1
        out_v[r, pl.ds(0, 16)] = (acc0[0] + acc0[1]) + (acc0[2] + acc0[3])
        out_v[r, pl.ds(16, 16)] = (acc1[0] + acc1[1]) + (acc1[2] + acc1[3])
        return carry

    lax.fori_loop(0, nrows, row, 0)


def _make_tec_body(F, T, EXTRA):
    def tec_body(p_h, in2_h, out_h,
                 pA, mA, oA, pB, mB, oB,
                 sia, sib, soa, sob):
        cid = lax.axis_index("c")
        sid = lax.axis_index("s")
        wid = sid * 2 + cid

        def start_in(t, dp, dm, sem):
            r0 = F + (t * NW + wid) * BLK
            pltpu.make_async_copy(p_h.at[pl.ds(r0, BLK), :], dp, sem).start()
            pltpu.make_async_copy(in2_h.at[pl.ds(r0, BLK), :], dm, sem).start()

        def wait_in(dp, dm, sem):
            pltpu.make_async_copy(p_h.at[pl.ds(0, BLK), :], dp, sem).wait()
            pltpu.make_async_copy(in2_h.at[pl.ds(0, BLK), :], dm, sem).wait()

        def start_out(t, src, sem):
            r0 = (t * NW + wid) * BLK
            pltpu.make_async_copy(src, out_h.at[pl.ds(r0, BLK), :], sem).start()

        def wait_out(src, sem):
            pltpu.make_async_copy(src, out_h.at[pl.ds(0, BLK), :], sem).wait()

        start_in(0, pA, mA, sia)

        def pair(i, carry):
            t0 = 2 * i
            wait_in(pA, mA, sia)
            start_in(t0 + 1, pB, mB, sib)

            @pl.when(i > 0)
            def _():
                wait_out(oA, soa)

            _row_compute(pA, mA, oA, BLK)
            start_out(t0, oA, soa)

            wait_in(pB, mB, sib)

            @pl.when(t0 + 2 < T)
            def _():
                start_in(t0 + 2, pA, mA, sia)

            @pl.when(i > 0)
            def _():
                wait_out(oB, sob)

            _row_compute(pB, mB, oB, BLK)
            start_out(t0 + 1, oB, sob)
            return carry

        lax.fori_loop(0, T // 2, pair, 0)
        wait_out(oA, soa)
        wait_out(oB, sob)

        for e in range(EXTRA):
            @pl.when(wid == (e % NW))
            def _():
                t = T * NW + e
                rg = F + t * BLK
                pltpu.sync_copy(p_h.at[pl.ds(rg, BLK), :], pA)
                pltpu.sync_copy(in2_h.at[pl.ds(rg, BLK), :], mA)
                _row_compute(pA, mA, oA, BLK)
                pltpu.sync_copy(oA, out_h.at[pl.ds(t * BLK, BLK), :])

    return tec_body


def _build_sc(F, nblk):
    nsc = nblk * BLK
    assert F % 8 == 0, F
    T = (nblk // NW) & ~1
    EXTRA = nblk - T * NW
    f32 = jnp.float32
    mesh = plsc.VectorSubcoreMesh(
        core_axis_name="c", subcore_axis_name="s", num_cores=2, num_subcores=16
    )
    return pl.kernel(
        _make_tec_body(F, T, EXTRA),
        out_type=jax.ShapeDtypeStruct((nsc, 32), f32),
        mesh=mesh,
        scratch_types=[
            pltpu.VMEM((BLK, 64), f32),
            pltpu.VMEM((BLK, 1024), f32),
            pltpu.VMEM((BLK, 32), f32),
            pltpu.VMEM((BLK, 64), f32),
            pltpu.VMEM((BLK, 1024), f32),
            pltpu.VMEM((BLK, 32), f32),
            pltpu.SemaphoreType.DMA,
            pltpu.SemaphoreType.DMA,
            pltpu.SemaphoreType.DMA,
            pltpu.SemaphoreType.DMA,
        ],
    )


def kernel(in0, in1, in2):
    n = in0.shape[0]
    f = FSPLIT * PB
    in0t = in0.T
    in1t = in1.T
    nblk_sc = (n - f) // BLK
    nblk1 = min(512, (nblk_sc // 2 // 16) * 16)
    nblk2 = nblk_sc - nblk1
    f2 = f + nblk1 * BLK
    pblk1 = (f2 - f + PB - 1) // PB
    pblk2 = (n - f2 + PB - 1) // PB
    p1 = _p_kernel(in0t, in1t, f // PB, pblk1)
    out_sc1 = _build_sc(f, nblk1)(p1, in2)
    p2 = _p_kernel(in0t, in1t, f2 // PB, pblk2)
    out_sc2 = _build_sc(f2, nblk2)(p2, in2)
    out_tc_t = _c_kernel(in0t, in1t, in2, FSPLIT)
    out_t = jnp.concatenate(
        [out_tc_t, out_sc1.T, out_sc2.T], axis=1)
    return out_t.T


# speedup vs baseline: 1.0586x; 1.0076x over previous
import jax
import jax.numpy as jnp
from jax import lax
from jax.experimental import pallas as pl
from jax.experimental.pallas import tpu as pltpu
from jax.experimental.pallas import tpu_sc as plsc

BLK = 40
NW = 32
PB = 1024
FSPLIT = 55


def _eye(n):
    i32 = jnp.int32
    return (lax.broadcasted_iota(i32, (n, n), 0)
            == lax.broadcasted_iota(i32, (n, n), 1)).astype(jnp.bfloat16)


def _split3(x):
    f32 = jnp.float32
    bf16 = jnp.bfloat16
    hi = x.astype(bf16)
    r1 = x - hi.astype(f32)
    mid = r1.astype(bf16)
    lo = (r1 - mid.astype(f32)).astype(bf16)
    return hi, mid, lo


def _p_block(in0t_ref, in1t_ref, p_ref):
    a = in0t_ref[...]
    b = in1t_ref[...]
    plo = a[0:32, :] * (0.5 * b[0:32, :] + 0.25 * b[32:64, :])
    phi = a[32:64, :] * (0.75 * b[0:32, :] - 0.25 * b[32:64, :])
    pt = jnp.concatenate([plo, phi], axis=0)
    p_ref[...] = pt.T


def _p_kernel(in0t, in1t, blk0, nblk):
    n = in0t.shape[1]
    return pl.pallas_call(
        _p_block,
        grid=(nblk,),
        in_specs=[
            pl.BlockSpec((64, PB), lambda i: (0, i + blk0)),
            pl.BlockSpec((64, PB), lambda i: (0, i + blk0)),
        ],
        out_specs=pl.BlockSpec((PB, 64), lambda i: (i + blk0, 0)),
        out_shape=jax.ShapeDtypeStruct((n, 64), jnp.float32),
    )(in0t, in1t)


def _c_block(in0t_ref, in1t_ref, m_ref, o_ref):
    i32 = jnp.int32
    a = in0t_ref[...]
    b = in1t_ref[...]
    plo = a[0:32, :] * (0.5 * b[0:32, :] + 0.25 * b[32:64, :])
    phi = a[32:64, :] * (0.75 * b[0:32, :] - 0.25 * b[32:64, :])
    pt = jnp.concatenate([plo, phi], axis=0)
    esel = (lax.broadcasted_iota(i32, (64, 1024), 1) // 16
            == lax.broadcasted_iota(i32, (64, 1024), 0)).astype(jnp.float32)
    q = lax.dot_general(pt, esel, (((0,), (0,)), ((), ())),
                        preferred_element_type=jnp.float32)
    t = q * m_ref[...]
    halves = []
    for h in range(2):
        x = t[:, 512 * h:512 * (h + 1)]
        w = 256
        while w >= 16:
            x = x[:, :w] + x[:, w:]
            w //= 2
        halves.append(x)
    o = jnp.concatenate(halves, axis=1)
    acc = None
    eye = _eye(32)
    for part in _split3(o):
        y = lax.dot_general(eye, part, (((1,), (1,)), ((), ())),
                            preferred_element_type=jnp.float32)
        acc = y if acc is None else acc + y
    o_ref[...] = acc


def _c_kernel(in0t, in1t, in2, nblk):
    return pl.pallas_call(
        _c_block,
        grid=(nblk,),
        in_specs=[
            pl.BlockSpec((64, PB), lambda i: (0, i)),
            pl.BlockSpec((64, PB), lambda i: (0, i)),
            pl.BlockSpec((PB, 1024), lambda i: (i, 0)),
        ],
        out_specs=pl.BlockSpec((32, PB), lambda i: (0, i)),
        out_shape=jax.ShapeDtypeStruct((32, nblk * PB), jnp.float32),
    )(in0t, in1t, in2)


def _row_compute(p_v, in2_v, out_v, nrows):
    def row(r, carry):
        p00 = p_v[r, pl.ds(0, 16)]
        p01 = p_v[r, pl.ds(16, 16)]
        p10 = p_v[r, pl.ds(32, 16)]
        p11 = p_v[r, pl.ds(48, 16)]

        acc0 = [None] * 4
        acc1 = [None] * 4
        for u in range(32):
            plo = p00[u] if u < 16 else p01[u - 16]
            phi = p10[u] if u < 16 else p11[u - 16]
            v0 = in2_v[r, pl.ds(u * 16, 16)] * plo
            v1 = in2_v[r, pl.ds(512 + u * 16, 16)] * phi
            j = u % 4
            if u < 4:
                acc0[j] = v0
                acc1[j] = v1
            else:
                acc0[j] = acc0[j] + v0
                acc1[j] = acc1[j] + v1
        out_v[r, pl.ds(0, 16)] = (acc0[0] + acc0[1]) + (acc0[2] + acc0[3])
        out_v[r, pl.ds(16, 16)] = (acc1[0] + acc1[1]) + (acc1[2] + acc1[3])
        return carry

    lax.fori_loop(0, nrows, row, 0)


def _make_tec_body(F, T, EXTRA):
    def tec_body(p_h, in2_h, out_h,
                 pA, mA, oA, pB, mB, oB,
                 sia, sib, soa, sob):
        cid = lax.axis_index("c")
        sid = lax.axis_index("s")
        wid = sid * 2 + cid

        def start_in(t, dp, dm, sem):
            r0 = F + (t * NW + wid) * BLK
            pltpu.make_async_copy(p_h.at[pl.ds(r0, BLK), :], dp, sem).start()
            pltpu.make_async_copy(in2_h.at[pl.ds(r0, BLK), :], dm, sem).start()

        def wait_in(dp, dm, sem):
            pltpu.make_async_copy(p_h.at[pl.ds(0, BLK), :], dp, sem).wait()
            pltpu.make_async_copy(in2_h.at[pl.ds(0, BLK), :], dm, sem).wait()

        def start_out(t, src, sem):
            r0 = (t * NW + wid) * BLK
            pltpu.make_async_copy(src, out_h.at[pl.ds(r0, BLK), :], sem).start()

        def wait_out(src, sem):
            pltpu.make_async_copy(src, out_h.at[pl.ds(0, BLK), :], sem).wait()

        start_in(0, pA, mA, sia)

        def pair(i, carry):
            t0 = 2 * i
            wait_in(pA, mA, sia)
            start_in(t0 + 1, pB, mB, sib)

            @pl.when(i > 0)
            def _():
                wait_out(oA, soa)

            _row_compute(pA, mA, oA, BLK)
            start_out(t0, oA, soa)

            wait_in(pB, mB, sib)

            @pl.when(t0 + 2 < T)
            def _():
                start_in(t0 + 2, pA, mA, sia)

            @pl.when(i > 0)
            def _():
                wait_out(oB, sob)

            _row_compute(pB, mB, oB, BLK)
            start_out(t0 + 1, oB, sob)
            return carry

        lax.fori_loop(0, T // 2, pair, 0)
        wait_out(oA, soa)
        wait_out(oB, sob)

        for e in range(EXTRA):
            @pl.when(wid == (e % NW))
            def _():
                t = T * NW + e
                rg = F + t * BLK
                pltpu.sync_copy(p_h.at[pl.ds(rg, BLK), :], pA)
                pltpu.sync_copy(in2_h.at[pl.ds(rg, BLK), :], mA)
                _row_compute(pA, mA, oA, BLK)
                pltpu.sync_copy(oA, out_h.at[pl.ds(t * BLK, BLK), :])

    return tec_body


def _build_sc(F, nblk):
    nsc = nblk * BLK
    assert F % 8 == 0, F
    T = (nblk // NW) & ~1
    EXTRA = nblk - T * NW
    f32 = jnp.float32
    mesh = plsc.VectorSubcoreMesh(
        core_axis_name="c", subcore_axis_name="s", num_cores=2, num_subcores=16
    )
    return pl.kernel(
        _make_tec_body(F, T, EXTRA),
        out_type=jax.ShapeDtypeStruct((nsc, 32), f32),
        mesh=mesh,
        scratch_types=[
            pltpu.VMEM((BLK, 64), f32),
            pltpu.VMEM((BLK, 1024), f32),
            pltpu.VMEM((BLK, 32), f32),
            pltpu.VMEM((BLK, 64), f32),
            pltpu.VMEM((BLK, 1024), f32),
            pltpu.VMEM((BLK, 32), f32),
            pltpu.SemaphoreType.DMA,
            pltpu.SemaphoreType.DMA,
            pltpu.SemaphoreType.DMA,
            pltpu.SemaphoreType.DMA,
        ],
    )


def kernel(in0, in1, in2):
    n = in0.shape[0]
    f = FSPLIT * PB
    in0t = in0.T
    in1t = in1.T
    nblk_sc = (n - f) // BLK
    nblk1 = min(384, (nblk_sc // 2 // 16) * 16)
    nblk2 = nblk_sc - nblk1
    f2 = f + nblk1 * BLK
    pblk1 = (f2 - f + PB - 1) // PB
    pblk2 = (n - f2 + PB - 1) // PB
    p1 = _p_kernel(in0t, in1t, f // PB, pblk1)
    out_sc1 = _build_sc(f, nblk1)(p1, in2)
    p2 = _p_kernel(in0t, in1t, f2 // PB, pblk2)
    out_sc2 = _build_sc(f2, nblk2)(p2, in2)
    out_tc_t = _c_kernel(in0t, in1t, in2, FSPLIT)
    out_t = jnp.concatenate(
        [out_tc_t, out_sc1.T, out_sc2.T], axis=1)
    return out_t.T
